# baseline (speedup 1.0000x reference)
import jax
import jax.numpy as jnp
from jax import lax
from jax.experimental import pallas as pl
from jax.experimental.pallas import tpu as pltpu

_B, _DM, _H, _W = 2, 96, 64, 64
_DI, _K, _N, _R = 192, 8, 16, 6
_L = _H * _W
_CP = _R + 2 * _N

_T = 1024
_NC = _L // _T
_TL = 512


def _inproj_kernel(x_ref, w_ref, xc_ref, z_ref):
    xz = jnp.dot(x_ref[0], w_ref[...],
                 preferred_element_type=jnp.float32)
    xc_ref[0] = xz[:, :_DI]
    zv = xz[:, _DI:]
    z_ref[0] = zv * jax.nn.sigmoid(zv)


def _softplus(x):
    return jnp.maximum(x, 0.0) + jnp.log1p(jnp.exp(-jnp.abs(x)))


def _scan_kernel(xs_ref, wp_ref, dtw_ref, bias_ref, a_ref, dv_ref,
                 out_ref, ea_ref, du_ref, bm_ref, hs_ref, h_ref):
    j = pl.program_id(1)
    rev = pl.program_id(0) % 2
    xs = xs_ref[0]
    x_dbl = jnp.dot(xs, wp_ref[0],
                    preferred_element_type=jnp.float32)
    dts = x_dbl[:, :_R]
    bmat = x_dbl[:, _R:_R + _N]
    cmat = x_dbl[:, _R + _N:]
    delta = _softplus(jnp.dot(dts, dtw_ref[0],
                              preferred_element_type=jnp.float32)
                      + bias_ref[0])
    ea_ref[...] = jnp.exp(delta[:, None, :] * a_ref[0][None])
    du_ref[...] = delta * xs
    bm_ref[...] = bmat[:, :, None]

    @pl.when(j == 0)
    def _():
        h_ref[...] = jnp.zeros_like(h_ref)

    def body(t, h):
        t2 = jnp.where(rev, _T - 1 - t, t)
        h = h * ea_ref[t2] + du_ref[t2] * bm_ref[t2]
        hs_ref[t2] = h
        return h

    h = lax.fori_loop(0, _T, body, h_ref[...], unroll=16)
    h_ref[...] = h
    y = jnp.sum(hs_ref[...] * cmat[:, :, None], axis=1)
    out_ref[0] = y + xs * dv_ref[0]


def _tail_kernel(y_ref, z_ref, lnw_ref, lnb_ref, wo_ref, out_ref):
    y = y_ref[0]
    mu = jnp.mean(y, axis=1, keepdims=True)
    d = y - mu
    var = jnp.mean(d * d, axis=1, keepdims=True)
    yn = d * lax.rsqrt(var + 1e-5) * lnw_ref[0] + lnb_ref[0]
    g = yn * z_ref[0]
    out_ref[0] = jnp.dot(g, wo_ref[...],
                         preferred_element_type=jnp.float32)


def kernel(x, w_in, conv_w, conv_b, x_proj_weight, dt_projs_weight,
           dt_projs_bias, A_logs, Ds, ln_w, ln_b, w_out):
    b = x.shape[0]
    x_t = jnp.swapaxes(x.reshape(b, _DM, _L), 1, 2)

    xc, z = pl.pallas_call(
        _inproj_kernel,
        grid=(b, _L // _TL),
        in_specs=[
            pl.BlockSpec((1, _TL, _DM), lambda i, j: (i, j, 0)),
            pl.BlockSpec((_DM, 2 * _DI), lambda i, j: (0, 0)),
        ],
        out_specs=[
            pl.BlockSpec((1, _TL, _DI), lambda i, j: (i, j, 0)),
            pl.BlockSpec((1, _TL, _DI), lambda i, j: (i, j, 0)),
        ],
        out_shape=[
            jax.ShapeDtypeStruct((b, _L, _DI), jnp.float32),
            jax.ShapeDtypeStruct((b, _L, _DI), jnp.float32),
        ],
        compiler_params=pltpu.CompilerParams(
            dimension_semantics=("parallel", "parallel")),
    )(x_t, jnp.swapaxes(w_in, 0, 1))

    xc = xc.reshape(b, _H, _W, _DI)
    w_nhwc = jnp.transpose(conv_w, (2, 3, 1, 0))
    xc = lax.conv_general_dilated(xc, w_nhwc, (1, 1), 'SAME',
                                  feature_group_count=_DI,
                                  dimension_numbers=('NHWC', 'HWIO', 'NHWC'))
    xc = jax.nn.silu(xc + conv_b[None, None, None, :])
    xc_t = xc.reshape(b, _L, _DI)

    lh = jnp.arange(_L)
    hp, wp = lh % _H, lh // _H
    g_tr = hp * _W + wp
    g_d = hp * _W + (hp + wp) % _W
    g_ad = hp * _W + (wp - hp) % _W
    x1t = jnp.take(xc_t, g_tr, axis=1)
    x4t = jnp.take(xc_t, g_d, axis=1)
    x5t = jnp.take(xc_t, g_ad, axis=1)
    src_all = jnp.stack([xc_t, x1t, x4t, x5t],
                        axis=1).reshape(b * 4, _L, _DI)

    wp_t = jnp.swapaxes(x_proj_weight, 1, 2)
    dtw_t = jnp.swapaxes(dt_projs_weight, 1, 2)
    bias2 = dt_projs_bias[:, None, :]
    a_t = jnp.swapaxes(-jnp.exp(A_logs).reshape(_K, _DI, _N), 1, 2)
    dv2 = Ds.reshape(_K, _DI)[:, None, :]

    def _ksel(i):
        s = (i % _K) // 2
        return (s // 2) * 4 + (s % 2) + 2 * (i % 2)

    def _src_map(i, j):
        jb = jnp.where(i % 2, _NC - 1 - j, j)
        return ((i // _K) * 4 + (i % _K) // 2, jb, 0)

    def _out_map(i, j):
        jb = jnp.where(i % 2, _NC - 1 - j, j)
        return (i, jb, 0)

    ys_t = pl.pallas_call(
        _scan_kernel,
        grid=(b * _K, _NC),
        in_specs=[
            pl.BlockSpec((1, _T, _DI), _src_map),
            pl.BlockSpec((1, _DI, _CP), lambda i, j: (_ksel(i), 0, 0)),
            pl.BlockSpec((1, _R, _DI), lambda i, j: (_ksel(i), 0, 0)),
            pl.BlockSpec((1, 1, _DI), lambda i, j: (_ksel(i), 0, 0)),
            pl.BlockSpec((1, _N, _DI), lambda i, j: (_ksel(i), 0, 0)),
            pl.BlockSpec((1, 1, _DI), lambda i, j: (_ksel(i), 0, 0)),
        ],
        out_specs=pl.BlockSpec((1, _T, _DI), _out_map),
        out_shape=jax.ShapeDtypeStruct((b * _K, _L, _DI), jnp.float32),
        scratch_shapes=[
            pltpu.VMEM((_T, _N, _DI), jnp.float32),
            pltpu.VMEM((_T, _DI), jnp.float32),
            pltpu.VMEM((_T, _N, 1), jnp.float32),
            pltpu.VMEM((_T, _N, _DI), jnp.float32),
            pltpu.VMEM((_N, _DI), jnp.float32),
        ],
        compiler_params=pltpu.CompilerParams(
            dimension_semantics=("parallel", "arbitrary")),
    )(src_all, wp_t, dtw_t, bias2, a_t, dv2)

    ysn = ys_t.reshape(b, _K, _L, _DI)
    p02 = ysn[:, 0] + ysn[:, 1]
    p13 = ysn[:, 2] + ysn[:, 3]
    p46 = ysn[:, 4] + ysn[:, 5]
    p57 = ysn[:, 6] + ysn[:, 7]
    ho, wo = lh // _W, lh % _W
    gi_tr = wo * _H + ho
    gi_d = ((wo - ho) % _W) * _H + ho
    gi_ad = ((wo + ho) % _W) * _H + ho
    y = (p02 + jnp.take(p13, gi_tr, axis=1)
         + jnp.take(p46, gi_d, axis=1)
         + jnp.take(p57, gi_ad, axis=1))

    out_t = pl.pallas_call(
        _tail_kernel,
        grid=(b, _L // _TL),
        in_specs=[
            pl.BlockSpec((1, _TL, _DI), lambda i, j: (i, j, 0)),
            pl.BlockSpec((1, _TL, _DI), lambda i, j: (i, j, 0)),
            pl.BlockSpec((1, _DI), lambda i, j: (0, 0)),
            pl.BlockSpec((1, _DI), lambda i, j: (0, 0)),
            pl.BlockSpec((_DI, _DM), lambda i, j: (0, 0)),
        ],
        out_specs=pl.BlockSpec((1, _TL, _DM), lambda i, j: (i, j, 0)),
        out_shape=jax.ShapeDtypeStruct((b, _L, _DM), jnp.float32),
        compiler_params=pltpu.CompilerParams(
            dimension_semantics=("parallel", "parallel")),
    )(y, z, ln_w[None, :], ln_b[None, :], jnp.swapaxes(w_out, 0, 1))

    return jnp.swapaxes(out_t, 1, 2).reshape(b, _DM, _H, _W)


# speedup vs baseline: 1.1672x; 1.1672x over previous
import jax
import jax.numpy as jnp
from jax import lax
from jax.experimental import pallas as pl
from jax.experimental.pallas import tpu as pltpu

_B, _DM, _H, _W = 2, 96, 64, 64
_DI, _K, _N, _R = 192, 8, 16, 6
_L = _H * _W
_CP = _R + 2 * _N

_T = 1024
_NC = _L // _T
_TL = 512


def _inproj_kernel(x_ref, w_ref, xc_ref, z_ref):
    xz = jnp.dot(x_ref[0], w_ref[...],
                 preferred_element_type=jnp.float32)
    xc_ref[0] = xz[:, :_DI]
    zv = xz[:, _DI:]
    z_ref[0] = zv * jax.nn.sigmoid(zv)


def _softplus(x):
    return jnp.maximum(x, 0.0) + jnp.log1p(jnp.exp(-jnp.abs(x)))


def _scan_kernel(xs_ref, wp_ref, dtw_ref, bias_ref, a_ref, dv_ref,
                 out_ref, ea_ref, bb_ref, hs_ref, h_ref):
    j = pl.program_id(1)
    rev = pl.program_id(0) % 2
    xs = xs_ref[0]
    x_dbl = jnp.dot(xs, wp_ref[0],
                    preferred_element_type=jnp.float32)
    dts = x_dbl[:, :_R]
    bmat = x_dbl[:, _R:_R + _N]
    cmat = x_dbl[:, _R + _N:]
    delta = _softplus(jnp.dot(dts, dtw_ref[0],
                              preferred_element_type=jnp.float32)
                      + bias_ref[0])
    ea_ref[...] = jnp.exp(delta[:, None, :] * a_ref[0][None])
    bb_ref[...] = (delta * xs)[:, None, :] * bmat[:, :, None]

    @pl.when(j == 0)
    def _():
        h_ref[...] = jnp.zeros_like(h_ref)

    def body(t, h):
        t2 = jnp.where(rev, _T - 1 - t, t)
        h = h * ea_ref[t2] + bb_ref[t2]
        hs_ref[t2] = h
        return h

    h = lax.fori_loop(0, _T, body, h_ref[...], unroll=16)
    h_ref[...] = h
    y = jnp.sum(hs_ref[...] * cmat[:, :, None], axis=1)
    out_ref[0] = y + xs * dv_ref[0]


def _tail_kernel(y_ref, z_ref, lnw_ref, lnb_ref, wo_ref, out_ref):
    y = y_ref[0]
    mu = jnp.mean(y, axis=1, keepdims=True)
    d = y - mu
    var = jnp.mean(d * d, axis=1, keepdims=True)
    yn = d * lax.rsqrt(var + 1e-5) * lnw_ref[0] + lnb_ref[0]
    g = yn * z_ref[0]
    out_ref[0] = jnp.dot(g, wo_ref[...],
                         preferred_element_type=jnp.float32)


def kernel(x, w_in, conv_w, conv_b, x_proj_weight, dt_projs_weight,
           dt_projs_bias, A_logs, Ds, ln_w, ln_b, w_out):
    b = x.shape[0]
    x_t = jnp.swapaxes(x.reshape(b, _DM, _L), 1, 2)

    xc, z = pl.pallas_call(
        _inproj_kernel,
        grid=(b, _L // _TL),
        in_specs=[
            pl.BlockSpec((1, _TL, _DM), lambda i, j: (i, j, 0)),
            pl.BlockSpec((_DM, 2 * _DI), lambda i, j: (0, 0)),
        ],
        out_specs=[
            pl.BlockSpec((1, _TL, _DI), lambda i, j: (i, j, 0)),
            pl.BlockSpec((1, _TL, _DI), lambda i, j: (i, j, 0)),
        ],
        out_shape=[
            jax.ShapeDtypeStruct((b, _L, _DI), jnp.float32),
            jax.ShapeDtypeStruct((b, _L, _DI), jnp.float32),
        ],
        compiler_params=pltpu.CompilerParams(
            dimension_semantics=("parallel", "parallel")),
    )(x_t, jnp.swapaxes(w_in, 0, 1))

    xc = xc.reshape(b, _H, _W, _DI)
    w_nhwc = jnp.transpose(conv_w, (2, 3, 1, 0))
    xc = lax.conv_general_dilated(xc, w_nhwc, (1, 1), 'SAME',
                                  feature_group_count=_DI,
                                  dimension_numbers=('NHWC', 'HWIO', 'NHWC'))
    xc = jax.nn.silu(xc + conv_b[None, None, None, :])
    xc_t = xc.reshape(b, _L, _DI)

    lh = jnp.arange(_L)
    hp, wp = lh % _H, lh // _H
    g_tr = hp * _W + wp
    g_d = hp * _W + (hp + wp) % _W
    g_ad = hp * _W + (wp - hp) % _W
    x1t = jnp.take(xc_t, g_tr, axis=1)
    x4t = jnp.take(xc_t, g_d, axis=1)
    x5t = jnp.take(xc_t, g_ad, axis=1)
    src_all = jnp.stack([xc_t, x1t, x4t, x5t],
                        axis=1).reshape(b * 4, _L, _DI)

    wp_t = jnp.swapaxes(x_proj_weight, 1, 2)
    dtw_t = jnp.swapaxes(dt_projs_weight, 1, 2)
    bias2 = dt_projs_bias[:, None, :]
    a_t = jnp.swapaxes(-jnp.exp(A_logs).reshape(_K, _DI, _N), 1, 2)
    dv2 = Ds.reshape(_K, _DI)[:, None, :]

    def _ksel(i):
        s = (i % _K) // 2
        return (s // 2) * 4 + (s % 2) + 2 * (i % 2)

    def _src_map(i, j):
        jb = jnp.where(i % 2, _NC - 1 - j, j)
        return ((i // _K) * 4 + (i % _K) // 2, jb, 0)

    def _out_map(i, j):
        jb = jnp.where(i % 2, _NC - 1 - j, j)
        return (i, jb, 0)

    ys_t = pl.pallas_call(
        _scan_kernel,
        grid=(b * _K, _NC),
        in_specs=[
            pl.BlockSpec((1, _T, _DI), _src_map),
            pl.BlockSpec((1, _DI, _CP), lambda i, j: (_ksel(i), 0, 0)),
            pl.BlockSpec((1, _R, _DI), lambda i, j: (_ksel(i), 0, 0)),
            pl.BlockSpec((1, 1, _DI), lambda i, j: (_ksel(i), 0, 0)),
            pl.BlockSpec((1, _N, _DI), lambda i, j: (_ksel(i), 0, 0)),
            pl.BlockSpec((1, 1, _DI), lambda i, j: (_ksel(i), 0, 0)),
        ],
        out_specs=pl.BlockSpec((1, _T, _DI), _out_map),
        out_shape=jax.ShapeDtypeStruct((b * _K, _L, _DI), jnp.float32),
        scratch_shapes=[
            pltpu.VMEM((_T, _N, _DI), jnp.float32),
            pltpu.VMEM((_T, _N, _DI), jnp.float32),
            pltpu.VMEM((_T, _N, _DI), jnp.float32),
            pltpu.VMEM((_N, _DI), jnp.float32),
        ],
        compiler_params=pltpu.CompilerParams(
            dimension_semantics=("parallel", "arbitrary")),
    )(src_all, wp_t, dtw_t, bias2, a_t, dv2)

    ysn = ys_t.reshape(b, _K, _L, _DI)
    p02 = ysn[:, 0] + ysn[:, 1]
    p13 = ysn[:, 2] + ysn[:, 3]
    p46 = ysn[:, 4] + ysn[:, 5]
    p57 = ysn[:, 6] + ysn[:, 7]
    ho, wo = lh // _W, lh % _W
    gi_tr = wo * _H + ho
    gi_d = ((wo - ho) % _W) * _H + ho
    gi_ad = ((wo + ho) % _W) * _H + ho
    y = (p02 + jnp.take(p13, gi_tr, axis=1)
         + jnp.take(p46, gi_d, axis=1)
         + jnp.take(p57, gi_ad, axis=1))

    out_t = pl.pallas_call(
        _tail_kernel,
        grid=(b, _L // _TL),
        in_specs=[
            pl.BlockSpec((1, _TL, _DI), lambda i, j: (i, j, 0)),
            pl.BlockSpec((1, _TL, _DI), lambda i, j: (i, j, 0)),
            pl.BlockSpec((1, _DI), lambda i, j: (0, 0)),
            pl.BlockSpec((1, _DI), lambda i, j: (0, 0)),
            pl.BlockSpec((_DI, _DM), lambda i, j: (0, 0)),
        ],
        out_specs=pl.BlockSpec((1, _TL, _DM), lambda i, j: (i, j, 0)),
        out_shape=jax.ShapeDtypeStruct((b, _L, _DM), jnp.float32),
        compiler_params=pltpu.CompilerParams(
            dimension_semantics=("parallel", "parallel")),
    )(y, z, ln_w[None, :], ln_b[None, :], jnp.swapaxes(w_out, 0, 1))

    return jnp.swapaxes(out_t, 1, 2).reshape(b, _DM, _H, _W)
